# baseline (speedup 1.0000x reference)
import functools

import jax
import jax.numpy as jnp
from jax import lax
from jax.experimental import pallas as pl
from jax.experimental.pallas import tpu as pltpu
from jax.experimental.pallas import tpu_sc as plsc

N_NODES = 10000
D = 128
NC = 2
NS = 16
N_TILES = NC * NS
E_CHUNK = 128
IDX_GRP = 40

N_PAD = 10240
ROWS_PER_TILE = N_PAD // NS
N_SUB = ROWS_PER_TILE // E_CHUNK

ROW_BLK = 1000


def _sc_agg_body(nc0, nc1, h_hbm, src_hbm, dst_hbm, z_hbm, out_hbm,
                 src_v, dst_v, rows0, rows1, acc_sh, g0, g1):
    c = lax.axis_index("c")
    s = lax.axis_index("s")
    n_my = jnp.where(c == 0, nc0, nc1)
    tile_base = jnp.where(c == 0, s * nc0, NS * nc0 + s * nc1)

    pltpu.sync_copy(z_hbm, rows0)
    for k in range(N_SUB):
        pltpu.sync_copy(
            rows0, acc_sh.at[pl.ds(s * ROWS_PER_TILE + k * E_CHUNK, E_CHUNK)])
    plsc.subcore_barrier()

    def gather_start(j, buf, sem):
        pltpu.make_async_copy(h_hbm.at[src_v.at[j]], buf, sem).start()

    def gather_wait(j, buf, sem):
        pltpu.make_async_copy(h_hbm.at[src_v.at[j]], buf, sem).wait()

    def scat(j, buf):
        pltpu.sync_copy(buf, acc_sh.at[dst_v.at[j]], add=True)

    @pl.loop(0, n_my // IDX_GRP)
    def _(g):
        base = pl.multiple_of(tile_base + g * IDX_GRP, 8)
        pltpu.sync_copy(src_hbm.at[pl.ds(base, IDX_GRP)], src_v)
        pltpu.sync_copy(dst_hbm.at[pl.ds(base, IDX_GRP)], dst_v)

        gather_start(0, rows0, g0)

        @pl.loop(0, IDX_GRP - 2, step=2)
        def _(jj):
            gather_wait(jj, rows0, g0)
            gather_start(jj + 1, rows1, g1)
            scat(jj, rows0)
            gather_wait(jj + 1, rows1, g1)
            gather_start(jj + 2, rows0, g0)
            scat(jj + 1, rows1)

        gather_wait(IDX_GRP - 2, rows0, g0)
        gather_start(IDX_GRP - 1, rows1, g1)
        scat(IDX_GRP - 2, rows0)
        gather_wait(IDX_GRP - 1, rows1, g1)
        scat(IDX_GRP - 1, rows1)

    plsc.subcore_barrier()
    for k in range(N_SUB):
        sl = pl.ds(s * ROWS_PER_TILE + k * E_CHUNK, E_CHUNK)
        pltpu.sync_copy(acc_sh.at[sl], rows0)
        pltpu.sync_copy(rows0, out_hbm.at[c, sl])


def _sc_aggregate(h, srcs2d, dsts2d, nc0, nc1):
    mesh = plsc.VectorSubcoreMesh(core_axis_name="c", subcore_axis_name="s")
    kern = pl.kernel(
        functools.partial(_sc_agg_body, nc0, nc1),
        out_type=jax.ShapeDtypeStruct((NC, N_PAD, D), jnp.float32),
        mesh=mesh,
        scratch_types=[
            pltpu.VMEM((IDX_GRP, E_CHUNK), jnp.int32),
            pltpu.VMEM((IDX_GRP, E_CHUNK), jnp.int32),
            pltpu.VMEM((E_CHUNK, D), jnp.float32),
            pltpu.VMEM((E_CHUNK, D), jnp.float32),
            pltpu.VMEM_SHARED((N_PAD, D), jnp.float32),
            pltpu.SemaphoreType.DMA,
            pltpu.SemaphoreType.DMA,
        ],
    )
    return kern(h, srcs2d, dsts2d, jnp.zeros((E_CHUNK, D), jnp.float32))


def _sc_hist_body(n_chunks, dst_hbm, z_hbm, ones_hbm, out_hbm,
                  dst_v, ones_v, hist_sh):
    c = lax.axis_index("c")
    s = lax.axis_index("s")
    w = c * NS + s

    pltpu.sync_copy(z_hbm, ones_v)
    for k in range(N_SUB):
        pltpu.sync_copy(
            ones_v, hist_sh.at[pl.ds(s * ROWS_PER_TILE + k * E_CHUNK, E_CHUNK)])
    pltpu.sync_copy(ones_hbm, ones_v)
    plsc.subcore_barrier()

    @pl.loop(0, n_chunks // IDX_GRP)
    def _(g):
        base = pl.multiple_of(w * n_chunks + g * IDX_GRP, 8)
        pltpu.sync_copy(dst_hbm.at[pl.ds(base, IDX_GRP)], dst_v)

        @pl.loop(0, IDX_GRP)
        def _(j):
            pltpu.sync_copy(ones_v, hist_sh.at[dst_v.at[j]], add=True)

    plsc.subcore_barrier()
    for k in range(N_SUB):
        sl = pl.ds(s * ROWS_PER_TILE + k * E_CHUNK, E_CHUNK)
        pltpu.sync_copy(hist_sh.at[sl], ones_v)
        pltpu.sync_copy(ones_v, out_hbm.at[c, sl])


def _sc_hist(dsts2d):
    n_chunks = dsts2d.shape[0] // N_TILES
    mesh = plsc.VectorSubcoreMesh(core_axis_name="c", subcore_axis_name="s")
    kern = pl.kernel(
        functools.partial(_sc_hist_body, n_chunks),
        out_type=jax.ShapeDtypeStruct((NC, N_PAD, D), jnp.float32),
        mesh=mesh,
        scratch_types=[
            pltpu.VMEM((IDX_GRP, E_CHUNK), jnp.int32),
            pltpu.VMEM((E_CHUNK, D), jnp.float32),
            pltpu.VMEM_SHARED((N_PAD, D), jnp.float32),
        ],
    )
    return kern(dsts2d, jnp.zeros((E_CHUNK, D), jnp.float32),
                jnp.ones((E_CHUNK, D), jnp.float32))


def _mm2_body(x_ref, wl_ref, wr_ref, h_ref, r_ref):
    xb = x_ref[...]
    h_ref[...] = jnp.dot(xb, wl_ref[...], preferred_element_type=jnp.float32)
    r_ref[...] = jnp.dot(xb, wr_ref[...], preferred_element_type=jnp.float32)


def _tc_transform(a, WlT, WrT):
    grid = (N_NODES // ROW_BLK,)
    return pl.pallas_call(
        _mm2_body,
        grid=grid,
        in_specs=[
            pl.BlockSpec((ROW_BLK, D), lambda i: (i, 0)),
            pl.BlockSpec((D, D), lambda i: (0, 0)),
            pl.BlockSpec((D, D), lambda i: (0, 0)),
        ],
        out_specs=[
            pl.BlockSpec((ROW_BLK, D), lambda i: (i, 0)),
            pl.BlockSpec((ROW_BLK, D), lambda i: (i, 0)),
        ],
        out_shape=[jax.ShapeDtypeStruct((N_NODES, D), jnp.float32)] * 2,
    )(a, WlT, WrT)


def _mid_body(agg_ref, hist_ref, r_ref, b_ref, wl_ref, wr_ref, h_ref, r2_ref):
    cnt = hist_ref[0][:, :1] + hist_ref[1][:, :1]
    inv = 1.0 / jnp.maximum(cnt, 1.0)
    mean = (agg_ref[0] + agg_ref[1]) * inv
    a1 = jnp.maximum(mean + r_ref[...] + b_ref[...], 0.0)
    h_ref[...] = jnp.dot(a1, wl_ref[...], preferred_element_type=jnp.float32)
    r2_ref[...] = jnp.dot(a1, wr_ref[...], preferred_element_type=jnp.float32)


def _tc_mid(agg, hist, r, b, WlT, WrT):
    grid = (N_NODES // ROW_BLK,)
    return pl.pallas_call(
        _mid_body,
        grid=grid,
        in_specs=[
            pl.BlockSpec((NC, ROW_BLK, D), lambda i: (0, i, 0)),
            pl.BlockSpec((NC, ROW_BLK, D), lambda i: (0, i, 0)),
            pl.BlockSpec((ROW_BLK, D), lambda i: (i, 0)),
            pl.BlockSpec((1, D), lambda i: (0, 0)),
            pl.BlockSpec((D, D), lambda i: (0, 0)),
            pl.BlockSpec((D, D), lambda i: (0, 0)),
        ],
        out_specs=[
            pl.BlockSpec((ROW_BLK, D), lambda i: (i, 0)),
            pl.BlockSpec((ROW_BLK, D), lambda i: (i, 0)),
        ],
        out_shape=[jax.ShapeDtypeStruct((N_NODES, D), jnp.float32)] * 2,
    )(agg, hist, r, b, WlT, WrT)


def _final_body(agg_ref, hist_ref, r_ref, b_ref, out_ref):
    cnt = hist_ref[0][:, :1] + hist_ref[1][:, :1]
    inv = 1.0 / jnp.maximum(cnt, 1.0)
    out_ref[...] = (agg_ref[0] + agg_ref[1]) * inv + r_ref[...] + b_ref[...]


def _tc_final(agg, hist, r, b):
    grid = (N_NODES // ROW_BLK,)
    return pl.pallas_call(
        _final_body,
        grid=grid,
        in_specs=[
            pl.BlockSpec((NC, ROW_BLK, D), lambda i: (0, i, 0)),
            pl.BlockSpec((NC, ROW_BLK, D), lambda i: (0, i, 0)),
            pl.BlockSpec((ROW_BLK, D), lambda i: (i, 0)),
            pl.BlockSpec((1, D), lambda i: (0, 0)),
        ],
        out_specs=pl.BlockSpec((ROW_BLK, D), lambda i: (i, 0)),
        out_shape=jax.ShapeDtypeStruct((N_NODES, D), jnp.float32),
    )(agg, hist, r, b)


def kernel(x, edge_index, W_l0, b_l0, W_r0, W_l1, b_l1, W_r1):
    e = edge_index.shape[1]
    n_chunks = -(-e // (N_TILES * E_CHUNK))
    n_chunks = -(-n_chunks // 8) * 8
    e_pad = n_chunks * N_TILES * E_CHUNK

    src = jnp.pad(edge_index[0].astype(jnp.int32), (0, e_pad - e))
    dst = jnp.pad(edge_index[1].astype(jnp.int32), (0, e_pad - e),
                  constant_values=N_NODES)
    srcs2d = src.reshape(-1, E_CHUNK)
    dsts2d = dst.reshape(-1, E_CHUNK)

    b0 = b_l0.reshape(1, D)
    b1 = b_l1.reshape(1, D)

    per_pair = 2 * n_chunks
    nc0 = (per_pair * 1 // 4) // IDX_GRP * IDX_GRP
    nc1 = per_pair - nc0

    h0, r0 = _tc_transform(x, W_l0.T, W_r0.T)
    agg0 = _sc_aggregate(h0, srcs2d, dsts2d, nc0, nc1)
    hist = _sc_hist(dsts2d)
    h1, r1 = _tc_mid(agg0, hist, r0, b0, W_l1.T, W_r1.T)
    agg1 = _sc_aggregate(h1, srcs2d, dsts2d, nc0, nc1)
    out = _tc_final(agg1, hist, r1, b1)
    return out


# speedup vs baseline: 1.1185x; 1.1185x over previous
import functools

import jax
import jax.numpy as jnp
from jax import lax
from jax.experimental import pallas as pl
from jax.experimental.pallas import tpu as pltpu
from jax.experimental.pallas import tpu_sc as plsc

N_NODES = 10000
D = 128
NC = 2
NS = 16
N_TILES = NC * NS
E_CHUNK = 128
IDX_GRP = 40

N_PAD = 10240
ROWS_PER_TILE = N_PAD // NS
N_SUB = ROWS_PER_TILE // E_CHUNK

ROW_BLK = 1000


def _sc_agg_body(nc0, nc1, h_hbm, src_hbm, dst_hbm, z_hbm, out_hbm,
                 src_v, dst_v, rows0, rows1, acc_sh, g0, g1):
    c = lax.axis_index("c")
    s = lax.axis_index("s")
    n_my = jnp.where(c == 0, nc0, nc1)
    tile_base = jnp.where(c == 0, s * nc0, NS * nc0 + s * nc1)

    pltpu.sync_copy(z_hbm, rows0)
    for k in range(N_SUB):
        pltpu.sync_copy(
            rows0, acc_sh.at[pl.ds(s * ROWS_PER_TILE + k * E_CHUNK, E_CHUNK)])
    plsc.subcore_barrier()

    def gather_start(j, buf, sem):
        pltpu.make_async_copy(h_hbm.at[src_v.at[j]], buf, sem).start()

    def gather_wait(j, buf, sem):
        pltpu.make_async_copy(h_hbm.at[src_v.at[j]], buf, sem).wait()

    def scat(j, buf):
        pltpu.sync_copy(buf, acc_sh.at[dst_v.at[j]], add=True)

    @pl.loop(0, n_my // IDX_GRP)
    def _(g):
        base = pl.multiple_of(tile_base + g * IDX_GRP, 8)
        pltpu.sync_copy(src_hbm.at[pl.ds(base, IDX_GRP)], src_v)
        pltpu.sync_copy(dst_hbm.at[pl.ds(base, IDX_GRP)], dst_v)

        gather_start(0, rows0, g0)

        @pl.loop(0, IDX_GRP - 2, step=2)
        def _(jj):
            gather_wait(jj, rows0, g0)
            gather_start(jj + 1, rows1, g1)
            scat(jj, rows0)
            gather_wait(jj + 1, rows1, g1)
            gather_start(jj + 2, rows0, g0)
            scat(jj + 1, rows1)

        gather_wait(IDX_GRP - 2, rows0, g0)
        gather_start(IDX_GRP - 1, rows1, g1)
        scat(IDX_GRP - 2, rows0)
        gather_wait(IDX_GRP - 1, rows1, g1)
        scat(IDX_GRP - 1, rows1)

    plsc.subcore_barrier()
    for k in range(N_SUB):
        sl = pl.ds(s * ROWS_PER_TILE + k * E_CHUNK, E_CHUNK)
        pltpu.sync_copy(acc_sh.at[sl], rows0)
        pltpu.sync_copy(rows0, out_hbm.at[c, sl])


def _sc_aggregate(h, srcs2d, dsts2d, nc0, nc1):
    mesh = plsc.VectorSubcoreMesh(core_axis_name="c", subcore_axis_name="s")
    kern = pl.kernel(
        functools.partial(_sc_agg_body, nc0, nc1),
        out_type=jax.ShapeDtypeStruct((NC, N_PAD, D), jnp.float32),
        mesh=mesh,
        scratch_types=[
            pltpu.VMEM((IDX_GRP, E_CHUNK), jnp.int32),
            pltpu.VMEM((IDX_GRP, E_CHUNK), jnp.int32),
            pltpu.VMEM((E_CHUNK, D), jnp.float32),
            pltpu.VMEM((E_CHUNK, D), jnp.float32),
            pltpu.VMEM_SHARED((N_PAD, D), jnp.float32),
            pltpu.SemaphoreType.DMA,
            pltpu.SemaphoreType.DMA,
        ],
    )
    return kern(h, srcs2d, dsts2d, jnp.zeros((E_CHUNK, D), jnp.float32))


def _sc_hist_body(n_chunks, dst_hbm, z_hbm, ones_hbm, out_hbm,
                  dst_v, ones_v, hist_sh):
    c = lax.axis_index("c")
    s = lax.axis_index("s")
    w = c * NS + s

    pltpu.sync_copy(z_hbm, ones_v)
    for k in range(N_SUB):
        pltpu.sync_copy(
            ones_v, hist_sh.at[pl.ds(s * ROWS_PER_TILE + k * E_CHUNK, E_CHUNK)])
    pltpu.sync_copy(ones_hbm, ones_v)
    plsc.subcore_barrier()

    @pl.loop(0, n_chunks // IDX_GRP)
    def _(g):
        base = pl.multiple_of(w * n_chunks + g * IDX_GRP, 8)
        pltpu.sync_copy(dst_hbm.at[pl.ds(base, IDX_GRP)], dst_v)

        @pl.loop(0, IDX_GRP)
        def _(j):
            pltpu.sync_copy(ones_v, hist_sh.at[dst_v.at[j]], add=True)

    plsc.subcore_barrier()
    for k in range(N_SUB):
        sl = pl.ds(s * ROWS_PER_TILE + k * E_CHUNK, E_CHUNK)
        pltpu.sync_copy(hist_sh.at[sl], ones_v)
        pltpu.sync_copy(ones_v, out_hbm.at[c, sl])


def _sc_hist(dsts2d):
    n_chunks = dsts2d.shape[0] // N_TILES
    mesh = plsc.VectorSubcoreMesh(core_axis_name="c", subcore_axis_name="s")
    kern = pl.kernel(
        functools.partial(_sc_hist_body, n_chunks),
        out_type=jax.ShapeDtypeStruct((NC, N_PAD, D), jnp.float32),
        mesh=mesh,
        scratch_types=[
            pltpu.VMEM((IDX_GRP, E_CHUNK), jnp.int32),
            pltpu.VMEM((E_CHUNK, D), jnp.float32),
            pltpu.VMEM_SHARED((N_PAD, D), jnp.float32),
        ],
    )
    return kern(dsts2d, jnp.zeros((E_CHUNK, D), jnp.float32),
                jnp.ones((E_CHUNK, D), jnp.float32))


def _mm2_body(x_ref, wl_ref, wr_ref, h_ref, r_ref):
    xb = x_ref[...]
    h_ref[...] = jnp.dot(xb, wl_ref[...], preferred_element_type=jnp.float32)
    r_ref[...] = jnp.dot(xb, wr_ref[...], preferred_element_type=jnp.float32)


def _tc_transform(a, WlT, WrT):
    grid = (N_NODES // ROW_BLK,)
    return pl.pallas_call(
        _mm2_body,
        grid=grid,
        in_specs=[
            pl.BlockSpec((ROW_BLK, D), lambda i: (i, 0)),
            pl.BlockSpec((D, D), lambda i: (0, 0)),
            pl.BlockSpec((D, D), lambda i: (0, 0)),
        ],
        out_specs=[
            pl.BlockSpec((ROW_BLK, D), lambda i: (i, 0)),
            pl.BlockSpec((ROW_BLK, D), lambda i: (i, 0)),
        ],
        out_shape=[jax.ShapeDtypeStruct((N_NODES, D), jnp.float32)] * 2,
    )(a, WlT, WrT)


def _mid_body(agg_ref, hist_ref, r_ref, b_ref, wl_ref, wr_ref, h_ref, r2_ref):
    cnt = hist_ref[0][:, :1] + hist_ref[1][:, :1]
    inv = 1.0 / jnp.maximum(cnt, 1.0)
    mean = (agg_ref[0] + agg_ref[1]) * inv
    a1 = jnp.maximum(mean + r_ref[...] + b_ref[...], 0.0)
    h_ref[...] = jnp.dot(a1, wl_ref[...], preferred_element_type=jnp.float32)
    r2_ref[...] = jnp.dot(a1, wr_ref[...], preferred_element_type=jnp.float32)


def _tc_mid(agg, hist, r, b, WlT, WrT):
    grid = (N_NODES // ROW_BLK,)
    return pl.pallas_call(
        _mid_body,
        grid=grid,
        in_specs=[
            pl.BlockSpec((NC, ROW_BLK, D), lambda i: (0, i, 0)),
            pl.BlockSpec((NC, ROW_BLK, D), lambda i: (0, i, 0)),
            pl.BlockSpec((ROW_BLK, D), lambda i: (i, 0)),
            pl.BlockSpec((1, D), lambda i: (0, 0)),
            pl.BlockSpec((D, D), lambda i: (0, 0)),
            pl.BlockSpec((D, D), lambda i: (0, 0)),
        ],
        out_specs=[
            pl.BlockSpec((ROW_BLK, D), lambda i: (i, 0)),
            pl.BlockSpec((ROW_BLK, D), lambda i: (i, 0)),
        ],
        out_shape=[jax.ShapeDtypeStruct((N_NODES, D), jnp.float32)] * 2,
    )(agg, hist, r, b, WlT, WrT)


def _final_body(agg_ref, hist_ref, r_ref, b_ref, out_ref):
    cnt = hist_ref[0][:, :1] + hist_ref[1][:, :1]
    inv = 1.0 / jnp.maximum(cnt, 1.0)
    out_ref[...] = (agg_ref[0] + agg_ref[1]) * inv + r_ref[...] + b_ref[...]


def _tc_final(agg, hist, r, b):
    grid = (N_NODES // ROW_BLK,)
    return pl.pallas_call(
        _final_body,
        grid=grid,
        in_specs=[
            pl.BlockSpec((NC, ROW_BLK, D), lambda i: (0, i, 0)),
            pl.BlockSpec((NC, ROW_BLK, D), lambda i: (0, i, 0)),
            pl.BlockSpec((ROW_BLK, D), lambda i: (i, 0)),
            pl.BlockSpec((1, D), lambda i: (0, 0)),
        ],
        out_specs=pl.BlockSpec((ROW_BLK, D), lambda i: (i, 0)),
        out_shape=jax.ShapeDtypeStruct((N_NODES, D), jnp.float32),
    )(agg, hist, r, b)


def kernel(x, edge_index, W_l0, b_l0, W_r0, W_l1, b_l1, W_r1):
    e = edge_index.shape[1]
    n_chunks = -(-e // (N_TILES * E_CHUNK))
    n_chunks = -(-n_chunks // 8) * 8
    e_pad = n_chunks * N_TILES * E_CHUNK

    src = jnp.pad(edge_index[0].astype(jnp.int32), (0, e_pad - e))
    dst = jnp.pad(edge_index[1].astype(jnp.int32), (0, e_pad - e),
                  constant_values=N_NODES)
    srcs2d = src.reshape(-1, E_CHUNK)
    dsts2d = dst.reshape(-1, E_CHUNK)

    b0 = b_l0.reshape(1, D)
    b1 = b_l1.reshape(1, D)

    per_pair = 2 * n_chunks
    nc0 = (per_pair * 2 // 4) // IDX_GRP * IDX_GRP
    nc1 = per_pair - nc0

    n_e0 = NS * nc0 * E_CHUNK
    src_shifted = jnp.concatenate([src[:n_e0], src[n_e0:] + N_NODES])
    srcs2d = src_shifted.reshape(-1, E_CHUNK)

    h0, r0 = _tc_transform(x, W_l0.T, W_r0.T)
    agg0 = _sc_aggregate(jnp.concatenate([h0, h0]), srcs2d, dsts2d, nc0, nc1)
    hist = _sc_hist(dsts2d)
    h1, r1 = _tc_mid(agg0, hist, r0, b0, W_l1.T, W_r1.T)
    agg1 = _sc_aggregate(jnp.concatenate([h1, h1]), srcs2d, dsts2d, nc0, nc1)
    out = _tc_final(agg1, hist, r1, b1)
    return out
